# baseline (speedup 1.0000x reference)
import functools

import jax
import jax.numpy as jnp
from jax import lax
from jax.experimental import pallas as pl
from jax.experimental.pallas import tpu as pltpu
from jax.experimental.pallas import tpu_sc as plsc

_N_FIELDS = 26
_VOCAB = 100000
_EMB_DIM = 32
_BATCH = 16384
_N_NUM = 13
_BN_EPS = 1e-5

_NC = 2
_NS = 16
_NW = _NC * _NS

_OUT_D = _N_FIELDS * _EMB_DIM + _N_NUM

_STRIP_B = 1024
_SPF = _BATCH // _STRIP_B
_N_STRIPS = _N_FIELDS * _SPF
_SPW = _N_STRIPS // _NW
_CHUNK = 128
_CPS = _STRIP_B // _CHUNK


def _sc_gather(tables_flat, idx):
    mesh = plsc.VectorSubcoreMesh(
        core_axis_name="c", subcore_axis_name="s",
        num_cores=_NC, num_subcores=_NS)

    @functools.partial(
        pl.kernel,
        out_type=jax.ShapeDtypeStruct((_N_STRIPS, _STRIP_B, _EMB_DIM),
                                      jnp.float32),
        mesh=mesh,
        scratch_types=[
            pltpu.VMEM((_SPW, _CPS, _CHUNK), jnp.int32),
            pltpu.VMEM((_STRIP_B, _EMB_DIM), jnp.float32),
            pltpu.VMEM((_STRIP_B, _EMB_DIM), jnp.float32),
            pltpu.SemaphoreType.DMA,
            pltpu.SemaphoreType.DMA,
        ],
        compiler_params=pltpu.CompilerParams(use_tc_tiling_on_sc=False),
    )
    def k(tbl_hbm, idx_hbm, out_hbm, idx_v, buf0, buf1, sem0, sem1):
        wid = lax.axis_index("c") * _NS + lax.axis_index("s")
        pltpu.sync_copy(idx_hbm.at[pl.ds(wid * _SPW, _SPW)], idx_v)

        bufs = (buf0, buf1)
        sems = (sem0, sem1)

        def fire(sl, p):
            for j in range(_CPS):
                pltpu.async_copy(
                    tbl_hbm.at[idx_v.at[sl, j]],
                    bufs[p].at[pl.ds(j * _CHUNK, _CHUNK)],
                    sems[p])

        def drain(sl, p):
            for j in range(_CPS):
                pltpu.make_async_copy(
                    tbl_hbm.at[idx_v.at[sl, j]],
                    bufs[p].at[pl.ds(j * _CHUNK, _CHUNK)],
                    sems[p]).wait()

        def wout(sl, p):
            pltpu.sync_copy(bufs[p], out_hbm.at[wid * _SPW + sl])

        fire(0, 0)

        def strip_pair(h, carry):
            s0 = 2 * h
            fire(s0 + 1, 1)
            drain(s0, 0)
            wout(s0, 0)
            fire(s0 + 2, 0)
            drain(s0 + 1, 1)
            wout(s0 + 1, 1)
            return carry

        lax.fori_loop(0, (_SPW - 1) // 2, strip_pair, 0)
        drain(_SPW - 1, 0)
        wout(_SPW - 1, 0)

    return k(tables_flat, idx)


def _bn_body(xt_ref, g_ref, b_ref, o_ref):
    x = xt_ref[...]
    mean = jnp.mean(x, axis=1, keepdims=True)
    xc = x - mean
    var = jnp.mean(xc * xc, axis=1, keepdims=True)
    o_ref[...] = xc * lax.rsqrt(var + _BN_EPS) * g_ref[...] + b_ref[...]


def _asm_body(strip_ref, cont_ref, eye_ref, o_ref):
    f = pl.program_id(0)

    @pl.when(f < _N_FIELDS)
    def _():
        o_ref[...] = lax.dot_general(
            eye_ref[...], strip_ref[0],
            (((1,), (1,)), ((), ())),
            preferred_element_type=jnp.float32)

    @pl.when(f == _N_FIELDS)
    def _():
        o_ref[...] = jnp.concatenate(
            [cont_ref[...],
             jnp.zeros((_EMB_DIM - _N_NUM, _STRIP_B), jnp.float32)], axis=0)


def _assemble(strips, cont_t):
    nf = _N_FIELDS + 1
    eye = jnp.eye(_EMB_DIM, dtype=jnp.float32)
    return pl.pallas_call(
        _asm_body,
        grid=(nf, _SPF),
        in_specs=[
            pl.BlockSpec((1, _STRIP_B, _EMB_DIM),
                         lambda f, b: (jnp.minimum(f, _N_FIELDS - 1) * _SPF + b,
                                       0, 0)),
            pl.BlockSpec((_N_NUM, _STRIP_B), lambda f, b: (0, b)),
            pl.BlockSpec((_EMB_DIM, _EMB_DIM), lambda f, b: (0, 0)),
        ],
        out_specs=pl.BlockSpec((_EMB_DIM, _STRIP_B), lambda f, b: (f, b)),
        out_shape=jax.ShapeDtypeStruct((_OUT_D, _BATCH), jnp.float32),
    )(strips, cont_t, eye)


def kernel(x_numerical, x_cat, tables, gamma, beta):
    idx = (x_cat.T.astype(jnp.int32)
           + jnp.arange(_N_FIELDS, dtype=jnp.int32)[:, None] * _VOCAB)
    idx = idx.reshape(_N_STRIPS, _CPS, _CHUNK)
    tables_flat = tables.reshape(_N_FIELDS * _VOCAB, _EMB_DIM)

    cont_t = pl.pallas_call(
        _bn_body,
        out_shape=jax.ShapeDtypeStruct((_N_NUM, _BATCH), jnp.float32),
    )(x_numerical.T, gamma.reshape(_N_NUM, 1), beta.reshape(_N_NUM, 1))

    strips = _sc_gather(tables_flat, idx)
    return _assemble(strips, cont_t).T


# speedup vs baseline: 1.2413x; 1.2413x over previous
import functools

import jax
import jax.numpy as jnp
from jax import lax
from jax.experimental import pallas as pl
from jax.experimental.pallas import tpu as pltpu
from jax.experimental.pallas import tpu_sc as plsc

_N_FIELDS = 26
_VOCAB = 100000
_EMB_DIM = 32
_BATCH = 16384
_N_NUM = 13
_BN_EPS = 1e-5

_NC = 2
_NS = 16
_NW = _NC * _NS

_OUT_D = _N_FIELDS * _EMB_DIM + _N_NUM
_EMB_D = _N_FIELDS * _EMB_DIM

_STRIP_B = 1024
_SPF = _BATCH // _STRIP_B
_N_STRIPS = _N_FIELDS * _SPF
_SPW = _N_STRIPS // _NW
_CHUNK = 128
_CPS = _STRIP_B // _CHUNK
_BPW = _BATCH // _NW


def _stats_body(x_ref, g_ref, b_ref, scale_ref, shift_ref, s_acc, q_acc):
    i = pl.program_id(0)

    @pl.when(i == 0)
    def _():
        s_acc[...] = jnp.zeros_like(s_acc)
        q_acc[...] = jnp.zeros_like(q_acc)

    x = x_ref[...]
    s_acc[...] += jnp.sum(x, axis=0)
    q_acc[...] += jnp.sum(x * x, axis=0)

    @pl.when(i == pl.num_programs(0) - 1)
    def _():
        n = float(_BATCH)
        mean = s_acc[...] / n
        var = q_acc[...] / n - mean * mean
        scale = g_ref[...] * lax.rsqrt(var + _BN_EPS)
        scale_ref[...] = scale
        shift_ref[...] = b_ref[...] - mean * scale


def _bn_stats(x_numerical, gamma, beta):
    grid = 8
    rows = _BATCH // grid
    return pl.pallas_call(
        _stats_body,
        grid=(grid,),
        in_specs=[
            pl.BlockSpec((rows, _N_NUM), lambda i: (i, 0)),
            pl.BlockSpec((_N_NUM,), lambda i: (0,)),
            pl.BlockSpec((_N_NUM,), lambda i: (0,)),
        ],
        out_specs=[
            pl.BlockSpec((_N_NUM,), lambda i: (0,)),
            pl.BlockSpec((_N_NUM,), lambda i: (0,)),
        ],
        out_shape=[
            jax.ShapeDtypeStruct((_N_NUM,), jnp.float32),
            jax.ShapeDtypeStruct((_N_NUM,), jnp.float32),
        ],
        scratch_shapes=[
            pltpu.VMEM((_N_NUM,), jnp.float32),
            pltpu.VMEM((_N_NUM,), jnp.float32),
        ],
    )(x_numerical, gamma, beta)


def _apply_body(x_ref, s_ref, t_ref, o_ref):
    o_ref[...] = x_ref[...] * s_ref[...] + t_ref[...]


def _bn_apply(x_numerical, scale, shift):
    grid = 8
    rows = _BATCH // grid
    return pl.pallas_call(
        _apply_body,
        grid=(grid,),
        in_specs=[
            pl.BlockSpec((rows, _N_NUM), lambda i: (i, 0)),
            pl.BlockSpec((_N_NUM,), lambda i: (0,)),
            pl.BlockSpec((_N_NUM,), lambda i: (0,)),
        ],
        out_specs=pl.BlockSpec((rows, _N_NUM), lambda i: (i, 0)),
        out_shape=jax.ShapeDtypeStruct((_BATCH, _N_NUM), jnp.float32),
    )(x_numerical, scale, shift)


def _sc_fused(tables_flat, idx, cont):
    mesh = plsc.VectorSubcoreMesh(
        core_axis_name="c", subcore_axis_name="s",
        num_cores=_NC, num_subcores=_NS)

    @functools.partial(
        pl.kernel,
        out_type=jax.ShapeDtypeStruct((_BATCH, _OUT_D), jnp.float32),
        mesh=mesh,
        scratch_types=[
            pltpu.VMEM((_SPW, _CPS, _CHUNK), jnp.int32),
            pltpu.VMEM((_STRIP_B, _EMB_DIM), jnp.float32),
            pltpu.VMEM((_STRIP_B, _EMB_DIM), jnp.float32),
            pltpu.VMEM((_BPW, _N_NUM), jnp.float32),
            pltpu.SemaphoreType.DMA,
            pltpu.SemaphoreType.DMA,
        ],
        compiler_params=pltpu.CompilerParams(use_tc_tiling_on_sc=False),
    )
    def k(tbl_hbm, idx_hbm, cont_hbm, out_hbm,
          idx_v, buf0, buf1, cvs, sem0, sem1):
        wid = lax.axis_index("c") * _NS + lax.axis_index("s")
        pltpu.sync_copy(idx_hbm.at[pl.ds(wid * _SPW, _SPW)], idx_v)

        bufs = (buf0, buf1)
        sems = (sem0, sem1)

        def fire(sl, p):
            for j in range(_CPS):
                pltpu.async_copy(
                    tbl_hbm.at[idx_v.at[sl, j]],
                    bufs[p].at[pl.ds(j * _CHUNK, _CHUNK)],
                    sems[p])

        def drain(sl, p):
            for j in range(_CPS):
                pltpu.make_async_copy(
                    tbl_hbm.at[idx_v.at[sl, j]],
                    bufs[p].at[pl.ds(j * _CHUNK, _CHUNK)],
                    sems[p]).wait()

        def wout(sl, p):
            s = wid * _SPW + sl
            f = s // _SPF
            b0 = (s - f * _SPF) * _STRIP_B
            pltpu.sync_copy(
                bufs[p],
                out_hbm.at[pl.ds(b0, _STRIP_B), pl.ds(f * _EMB_DIM, _EMB_DIM)])

        fire(0, 0)

        base = wid * _BPW
        pltpu.sync_copy(cont_hbm.at[pl.ds(base, _BPW)], cvs)
        pltpu.sync_copy(cvs, out_hbm.at[pl.ds(base, _BPW), pl.ds(_EMB_D, _N_NUM)])

        def strip_pair(h, carry):
            s0 = 2 * h
            fire(s0 + 1, 1)
            drain(s0, 0)
            wout(s0, 0)
            fire(s0 + 2, 0)
            drain(s0 + 1, 1)
            wout(s0 + 1, 1)
            return carry

        lax.fori_loop(0, (_SPW - 1) // 2, strip_pair, 0)
        drain(_SPW - 1, 0)
        wout(_SPW - 1, 0)

    return k(tables_flat, idx, cont)


def kernel(x_numerical, x_cat, tables, gamma, beta):
    idx = (x_cat.T.astype(jnp.int32)
           + jnp.arange(_N_FIELDS, dtype=jnp.int32)[:, None] * _VOCAB)
    idx = idx.reshape(_N_STRIPS, _CPS, _CHUNK)
    tables_flat = tables.reshape(_N_FIELDS * _VOCAB, _EMB_DIM)

    scale, shift = _bn_stats(x_numerical, gamma, beta)
    cont = _bn_apply(x_numerical, scale, shift)

    return _sc_fused(tables_flat, idx, cont)


# speedup vs baseline: 1.3217x; 1.0648x over previous
import functools

import jax
import jax.numpy as jnp
from jax import lax
from jax.experimental import pallas as pl
from jax.experimental.pallas import tpu as pltpu
from jax.experimental.pallas import tpu_sc as plsc

_N_FIELDS = 26
_VOCAB = 100000
_EMB_DIM = 32
_BATCH = 16384
_N_NUM = 13
_BN_EPS = 1e-5

_NC = 2
_NS = 16
_NW = _NC * _NS

_OUT_D = _N_FIELDS * _EMB_DIM + _N_NUM

_STRIP_B = 1024
_SPF = _BATCH // _STRIP_B
_N_STRIPS = _N_FIELDS * _SPF
_SPW = _N_STRIPS // _NW
_CHUNK = 128
_CPS = _STRIP_B // _CHUNK
_LANES = 128
_ROWS_PER_F = _STRIP_B * _EMB_DIM // _LANES


def _sc_gather(tables_flat, idx):
    mesh = plsc.VectorSubcoreMesh(
        core_axis_name="c", subcore_axis_name="s",
        num_cores=_NC, num_subcores=_NS)

    @functools.partial(
        pl.kernel,
        out_type=jax.ShapeDtypeStruct((_N_STRIPS, _STRIP_B, _EMB_DIM),
                                      jnp.float32),
        mesh=mesh,
        scratch_types=[
            pltpu.VMEM((_SPW, _CPS, _CHUNK), jnp.int32),
            pltpu.VMEM((_STRIP_B, _EMB_DIM), jnp.float32),
            pltpu.VMEM((_STRIP_B, _EMB_DIM), jnp.float32),
            pltpu.SemaphoreType.DMA,
            pltpu.SemaphoreType.DMA,
        ],
        compiler_params=pltpu.CompilerParams(use_tc_tiling_on_sc=False),
    )
    def k(tbl_hbm, idx_hbm, out_hbm, idx_v, buf0, buf1, sem0, sem1):
        wid = lax.axis_index("c") * _NS + lax.axis_index("s")
        pltpu.sync_copy(idx_hbm.at[pl.ds(wid * _SPW, _SPW)], idx_v)

        bufs = (buf0, buf1)
        sems = (sem0, sem1)

        def fire(sl, p):
            for j in range(_CPS):
                pltpu.async_copy(
                    tbl_hbm.at[idx_v.at[sl, j]],
                    bufs[p].at[pl.ds(j * _CHUNK, _CHUNK)],
                    sems[p])

        def drain(sl, p):
            for j in range(_CPS):
                pltpu.make_async_copy(
                    tbl_hbm.at[idx_v.at[sl, j]],
                    bufs[p].at[pl.ds(j * _CHUNK, _CHUNK)],
                    sems[p]).wait()

        def wout(sl, p):
            pltpu.sync_copy(bufs[p], out_hbm.at[wid * _SPW + sl])

        fire(0, 0)

        def strip_pair(h, carry):
            s0 = 2 * h
            fire(s0 + 1, 1)
            drain(s0, 0)
            wout(s0, 0)
            fire(s0 + 2, 0)
            drain(s0 + 1, 1)
            wout(s0 + 1, 1)
            return carry

        lax.fori_loop(0, (_SPW - 1) // 2, strip_pair, 0)
        drain(_SPW - 1, 0)
        wout(_SPW - 1, 0)

    return k(tables_flat, idx)


def _bn_body(xt_ref, g_ref, b_ref, o_ref):
    x = xt_ref[...]
    mean = jnp.mean(x, axis=1, keepdims=True)
    xc = x - mean
    var = jnp.mean(xc * xc, axis=1, keepdims=True)
    o_ref[...] = xc * lax.rsqrt(var + _BN_EPS) * g_ref[...] + b_ref[...]


def _asm_body(y_ref, cont_ref, o_ref):
    for f in range(_N_FIELDS):
        yf = y_ref[pl.ds(f * _ROWS_PER_F, _ROWS_PER_F), :]
        t = jnp.transpose(yf, (1, 0))
        for q in range(_STRIP_B // _ROWS_PER_F):
            o_ref[pl.ds(f * _EMB_DIM, _EMB_DIM),
                  pl.ds(q * _ROWS_PER_F, _ROWS_PER_F)] = (
                t[q * _EMB_DIM:(q + 1) * _EMB_DIM, :])
    o_ref[pl.ds(_N_FIELDS * _EMB_DIM, _N_NUM), :] = cont_ref[...]


def _assemble(strips_flat, cont_t):
    return pl.pallas_call(
        _asm_body,
        grid=(_SPF,),
        in_specs=[
            pl.BlockSpec((None, _N_FIELDS * _ROWS_PER_F, _LANES),
                         lambda b: (b, 0, 0)),
            pl.BlockSpec((_N_NUM, _STRIP_B), lambda b: (0, b)),
        ],
        out_specs=pl.BlockSpec((_OUT_D, _STRIP_B), lambda b: (0, b)),
        out_shape=jax.ShapeDtypeStruct((_OUT_D, _BATCH), jnp.float32),
    )(strips_flat, cont_t)


def kernel(x_numerical, x_cat, tables, gamma, beta):
    idx = (x_cat.T.astype(jnp.int32)
           + jnp.arange(_N_FIELDS, dtype=jnp.int32)[:, None] * _VOCAB)
    idx = idx.reshape(_N_FIELDS, _SPF, 4, _ROWS_PER_F)
    idx = idx.transpose(1, 0, 3, 2)
    idx = idx.reshape(_N_STRIPS, _CPS, _CHUNK)
    tables_flat = tables.reshape(_N_FIELDS * _VOCAB, _EMB_DIM)

    cont_t = pl.pallas_call(
        _bn_body,
        out_shape=jax.ShapeDtypeStruct((_N_NUM, _BATCH), jnp.float32),
    )(x_numerical.T, gamma.reshape(_N_NUM, 1), beta.reshape(_N_NUM, 1))

    strips = _sc_gather(tables_flat, idx)
    strips_flat = strips.reshape(_SPF, _N_FIELDS * _ROWS_PER_F, _LANES)
    return _assemble(strips_flat, cont_t).T
